# baseline (device time: 38294 ns/iter reference)
import jax
import jax.numpy as jnp
from jax import lax
from jax.experimental import pallas as pl
from jax.experimental.pallas import tpu as pltpu

N_DEV = 4
B, SQ, D = 2, 256, 512
H_LOCAL, DH = 4, 64
ROWS = B * SQ
Q4 = ROWS // N_DEV
EPS = 1e-5
F32 = jnp.float32
BF16 = jnp.bfloat16


def kernel(x, Wq, Wk, Wv, Wo, t_emb, W_mod, W_ff1, W_ff2):
    def body(
        x_hbm, wq_hbm, wk_hbm, wv_hbm, wo_hbm, temb_hbm, wmod_hbm,
        wff1_hbm, wff2_hbm, out_ref,
        xv_ref, wq_ref, wk_ref, wv_ref, wo_ref, temb_ref, wmod_ref,
        wff1_ref, wff2_ref,
        x0q_ref,
        pf_ref,
        p1_ref,
        rs1_ref,
        x1ag_ref,
        p2_ref,
        rs2_ref,
        red2_ref,
        load_sems,
        send_sems, recv_sems,
    ):
        my = lax.axis_index("i")

        hbm = [x_hbm, temb_hbm, wmod_hbm, wq_hbm, wk_hbm, wv_hbm,
               wo_hbm, wff1_hbm, wff2_hbm]
        vmem = [xv_ref, temb_ref, wmod_ref, wq_ref, wk_ref, wv_ref,
                wo_ref, wff1_ref, wff2_ref]
        loads = []
        for i, (h, v) in enumerate(zip(hbm, vmem)):
            c = pltpu.make_async_copy(h, v, load_sems.at[i])
            c.start()
            loads.append(c)

        barrier_sem = pltpu.get_barrier_semaphore()
        for off in (1, 2, 3):
            pl.semaphore_signal(
                barrier_sem, inc=1,
                device_id=(lax.rem(my + off, N_DEV),),
                device_id_type=pl.DeviceIdType.MESH,
            )
        pl.semaphore_wait(barrier_sem, N_DEV - 1)

        def mk_rdma(phase, off, src, dst, dev):
            return pltpu.make_async_remote_copy(
                src_ref=src,
                dst_ref=dst,
                send_sem=send_sems.at[phase * 3 + off - 1],
                recv_sem=recv_sems.at[phase * 3 + off - 1],
                device_id=(dev,),
                device_id_type=pl.DeviceIdType.MESH,
            )

        def exchange(phase, make_src, make_dst):
            rdmas = []
            for off in (1, 2, 3):
                dev = lax.rem(my + off, N_DEV)
                rdma = mk_rdma(phase, off, make_src(off, dev),
                               make_dst(off, dev), dev)
                rdma.start()
                rdmas.append(rdma)
            return rdmas

        def store_quarters(ref, val):
            for q in range(N_DEV):
                ref[q] = val[q * Q4 : (q + 1) * Q4, :].astype(ref.dtype)

        def bsel(pair, qidx):
            return jnp.where(qidx >= 2, pair[1], pair[0])

        loads[1].wait()
        loads[2].wait()
        mod = []
        for b in range(B):
            mb = jnp.dot(
                temb_ref[b : b + 1, :], wmod_ref[...],
                preferred_element_type=F32,
            )
            mod.append([mb[:, i * D : (i + 1) * D] for i in range(6)])
        sa_, sha_, ga_, sm_, shm_, gm_ = (
            [mod[0][i], mod[1][i]] for i in range(6)
        )

        loads[0].wait()
        for q in range(N_DEV):
            x0q_ref[q] = xv_ref[q // 2][(q % 2) * Q4 : (q % 2 + 1) * Q4, :]

        def ln_mod(h, scale, shift):
            m = jnp.mean(h, axis=-1, keepdims=True)
            v = jnp.mean((h - m) * (h - m), axis=-1, keepdims=True)
            return ((h - m) * lax.rsqrt(v + EPS)) * (1.0 + scale) + shift

        loads[3].wait()
        loads[4].wait()
        loads[5].wait()
        wq_b = wq_ref[...].astype(BF16)
        wk_b = wk_ref[...].astype(BF16)
        wv_b = wv_ref[...].astype(BF16)
        attn_parts = []
        for b in range(B):
            xm = ln_mod(xv_ref[b], sa_[b], sha_[b]).astype(BF16)
            q = jnp.dot(xm, wq_b, preferred_element_type=F32).astype(BF16)
            k = jnp.dot(xm, wk_b, preferred_element_type=F32).astype(BF16)
            v = jnp.dot(xm, wv_b, preferred_element_type=F32).astype(BF16)
            outs = []
            for h in range(H_LOCAL):
                sl = slice(h * DH, (h + 1) * DH)
                s = lax.dot_general(
                    q[:, sl], k[:, sl], (((1,), (1,)), ((), ())),
                    preferred_element_type=F32,
                ) * 0.125
                mx = jnp.max(s, axis=-1, keepdims=True)
                p = jnp.exp(s - mx)
                l = jnp.sum(p, axis=-1, keepdims=True)
                outs.append(
                    jnp.dot(
                        p.astype(BF16), v[:, sl], preferred_element_type=F32
                    ) / l
                )
            o = jnp.concatenate(outs, axis=1).astype(BF16)
            if b == 0:
                loads[6].wait()
                wo_b = wo_ref[...].astype(BF16)
            attn_parts.append(jnp.dot(o, wo_b, preferred_element_type=F32))
        attn_partial = jnp.concatenate(attn_parts, axis=0)
        store_quarters(pf_ref, attn_partial)
        store_quarters(p1_ref, attn_partial)

        rs1 = exchange(
            0,
            lambda off, dev: p1_ref.at[dev],
            lambda off, dev: rs1_ref.at[off - 1],
        )
        for r in rs1:
            r.wait()
        attn_my = pf_ref[my]
        for i in range(N_DEV - 1):
            attn_my = attn_my + rs1_ref[i].astype(F32)

        x1_my = x0q_ref[my] + bsel(ga_, my) * attn_my
        x1ag_ref[my] = x1_my.astype(BF16)
        ag1 = exchange(
            1,
            lambda off, dev: x1ag_ref.at[my],
            lambda off, dev: x1ag_ref.at[my],
        )

        loads[7].wait()
        loads[8].wait()
        wff1_b = wff1_ref[...].astype(BF16)
        wff2_b = wff2_ref[...].astype(BF16)

        def ffn_block(x1_blk, qidx):
            xm2 = ln_mod(x1_blk, bsel(sm_, qidx), bsel(shm_, qidx))
            hb = jnp.dot(xm2.astype(BF16), wff1_b, preferred_element_type=F32)
            hb = hb / (1.0 + jnp.exp(-hb))
            return jnp.dot(hb.astype(BF16), wff2_b, preferred_element_type=F32)

        ffn_my = ffn_block(x1_my, my)

        rs2 = []
        for off in (1, 2, 3):
            ag1[off - 1].wait_recv()
            qidx = lax.rem(my - off + N_DEV, N_DEV)
            fblk = ffn_block(x1ag_ref[qidx].astype(F32), qidx)
            p2_ref[qidx] = fblk.astype(BF16)
            off_send = N_DEV - off
            r = mk_rdma(
                2, off_send, p2_ref.at[qidx], rs2_ref.at[off_send - 1], qidx
            )
            r.start()
            rs2.append(r)

        for r in rs2:
            r.wait_recv()
        total2 = ffn_my
        for i in range(N_DEV - 1):
            total2 = total2 + rs2_ref[i].astype(F32)
        out_my = x1_my + bsel(gm_, my) * total2
        red2_ref[my] = out_my.astype(BF16)
        ag2 = exchange(
            3,
            lambda off, dev: red2_ref.at[my],
            lambda off, dev: red2_ref.at[my],
        )
        for r in ag2:
            r.wait_recv()

        for b in range(B):
            out_ref[b, :, :] = jnp.concatenate(
                [red2_ref[2 * b].astype(F32), red2_ref[2 * b + 1].astype(F32)],
                axis=0,
            )

        for r in ag1:
            r.wait_send()
        for r in rs2:
            r.wait_send()
        for r in ag2:
            r.wait_send()

    return pl.pallas_call(
        body,
        out_shape=jax.ShapeDtypeStruct((B, SQ, D), jnp.float32),
        in_specs=[pl.BlockSpec(memory_space=pl.ANY)] * 9,
        out_specs=pl.BlockSpec(memory_space=pltpu.VMEM),
        scratch_shapes=[
            pltpu.VMEM((B, SQ, D), F32),
            pltpu.VMEM((D, 256), F32),
            pltpu.VMEM((D, 256), F32),
            pltpu.VMEM((D, 256), F32),
            pltpu.VMEM((256, D), F32),
            pltpu.VMEM((B, 128), F32),
            pltpu.VMEM((128, 6 * D), F32),
            pltpu.VMEM((D, D), F32),
            pltpu.VMEM((D, D), F32),
            pltpu.VMEM((N_DEV, Q4, D), F32),
            pltpu.VMEM((N_DEV, Q4, D), F32),
            pltpu.VMEM((N_DEV, Q4, D), BF16),
            pltpu.VMEM((3, Q4, D), BF16),
            pltpu.VMEM((N_DEV, Q4, D), BF16),
            pltpu.VMEM((N_DEV, Q4, D), BF16),
            pltpu.VMEM((3, Q4, D), BF16),
            pltpu.VMEM((N_DEV, Q4, D), BF16),
            pltpu.SemaphoreType.DMA((9,)),
            pltpu.SemaphoreType.DMA((12,)),
            pltpu.SemaphoreType.DMA((12,)),
        ],
        compiler_params=pltpu.CompilerParams(collective_id=0),
    )(x, Wq, Wk, Wv, Wo, t_emb, W_mod, W_ff1, W_ff2)


# device time: 38276 ns/iter; 1.0005x vs baseline; 1.0005x over previous
import jax
import jax.numpy as jnp
from jax import lax
from jax.experimental import pallas as pl
from jax.experimental.pallas import tpu as pltpu

N_DEV = 4
B, SQ, D = 2, 256, 512
H_LOCAL, DH = 4, 64
ROWS = B * SQ
Q4 = ROWS // N_DEV
EPS = 1e-5
F32 = jnp.float32
BF16 = jnp.bfloat16


def kernel(x, Wq, Wk, Wv, Wo, t_emb, W_mod, W_ff1, W_ff2):
    def body(
        x_hbm, wq_hbm, wk_hbm, wv_hbm, wo_hbm, temb_hbm, wmod_hbm,
        wff1_hbm, wff2_hbm, out_ref,
        xv_ref, wq_ref, wk_ref, wv_ref, wo_ref, temb_ref, wmod_ref,
        wff1_ref, wff2_ref,
        x0q_ref,
        pf_ref,
        p1_ref,
        rs1_ref,
        x1ag_ref,
        p2_ref,
        rs2_ref,
        red2_ref,
        load_sems,
        send_sems, recv_sems,
    ):
        my = lax.axis_index("i")

        hbm = [x_hbm, temb_hbm, wmod_hbm, wq_hbm, wk_hbm, wv_hbm,
               wo_hbm, wff1_hbm, wff2_hbm]
        vmem = [xv_ref, temb_ref, wmod_ref, wq_ref, wk_ref, wv_ref,
                wo_ref, wff1_ref, wff2_ref]
        loads = []
        for i, (h, v) in enumerate(zip(hbm, vmem)):
            c = pltpu.make_async_copy(h, v, load_sems.at[i])
            c.start()
            loads.append(c)

        barrier_sem = pltpu.get_barrier_semaphore()
        for off in (1, 2, 3):
            pl.semaphore_signal(
                barrier_sem, inc=1,
                device_id=(lax.rem(my + off, N_DEV),),
                device_id_type=pl.DeviceIdType.MESH,
            )
        pl.semaphore_wait(barrier_sem, N_DEV - 1)

        def mk_rdma(phase, off, src, dst, dev):
            return pltpu.make_async_remote_copy(
                src_ref=src,
                dst_ref=dst,
                send_sem=send_sems.at[phase * 3 + off - 1],
                recv_sem=recv_sems.at[phase * 3 + off - 1],
                device_id=(dev,),
                device_id_type=pl.DeviceIdType.MESH,
            )

        def exchange(phase, make_src, make_dst):
            rdmas = []
            for off in (1, 2, 3):
                dev = lax.rem(my + off, N_DEV)
                rdma = mk_rdma(phase, off, make_src(off, dev),
                               make_dst(off, dev), dev)
                rdma.start()
                rdmas.append(rdma)
            return rdmas

        def store_quarters(ref, val):
            for q in range(N_DEV):
                ref[q] = val[q * Q4 : (q + 1) * Q4, :].astype(ref.dtype)

        def bsel(pair, qidx):
            return jnp.where(qidx >= 2, pair[1], pair[0])

        loads[1].wait()
        loads[2].wait()
        mod = []
        for b in range(B):
            mb = jnp.dot(
                temb_ref[b : b + 1, :], wmod_ref[...],
                preferred_element_type=F32,
            )
            mod.append([mb[:, i * D : (i + 1) * D] for i in range(6)])
        sa_, sha_, ga_, sm_, shm_, gm_ = (
            [mod[0][i], mod[1][i]] for i in range(6)
        )

        loads[0].wait()
        for q in range(N_DEV):
            x0q_ref[q] = xv_ref[q // 2][(q % 2) * Q4 : (q % 2 + 1) * Q4, :]

        def ln_mod(h, scale, shift):
            m = jnp.mean(h, axis=-1, keepdims=True)
            v = jnp.mean((h - m) * (h - m), axis=-1, keepdims=True)
            return ((h - m) * lax.rsqrt(v + EPS)) * (1.0 + scale) + shift

        loads[3].wait()
        loads[4].wait()
        loads[5].wait()
        wq_b = wq_ref[...].astype(BF16)
        wk_b = wk_ref[...].astype(BF16)
        wv_b = wv_ref[...].astype(BF16)
        attn_parts = []
        for b in range(B):
            xm = ln_mod(xv_ref[b], sa_[b], sha_[b]).astype(BF16)
            q = jnp.dot(xm, wq_b, preferred_element_type=F32).astype(BF16)
            k = jnp.dot(xm, wk_b, preferred_element_type=F32).astype(BF16)
            v = jnp.dot(xm, wv_b, preferred_element_type=F32).astype(BF16)
            outs = []
            for h in range(H_LOCAL):
                sl = slice(h * DH, (h + 1) * DH)
                s = lax.dot_general(
                    q[:, sl], k[:, sl], (((1,), (1,)), ((), ())),
                    preferred_element_type=F32,
                ) * 0.125
                mx = jnp.max(s, axis=-1, keepdims=True)
                p = jnp.exp(s - mx)
                l = jnp.sum(p, axis=-1, keepdims=True)
                outs.append(
                    jnp.dot(
                        p.astype(BF16), v[:, sl], preferred_element_type=F32
                    ) / l
                )
            o = jnp.concatenate(outs, axis=1).astype(BF16)
            if b == 0:
                loads[6].wait()
                wo_b = wo_ref[...].astype(BF16)
            attn_parts.append(jnp.dot(o, wo_b, preferred_element_type=F32))
        attn_partial = jnp.concatenate(attn_parts, axis=0)
        store_quarters(pf_ref, attn_partial)
        store_quarters(p1_ref, attn_partial)

        rs1 = exchange(
            0,
            lambda off, dev: p1_ref.at[dev],
            lambda off, dev: rs1_ref.at[off - 1],
        )
        for r in rs1:
            r.wait()
        attn_my = pf_ref[my]
        for i in range(N_DEV - 1):
            attn_my = attn_my + rs1_ref[i].astype(F32)

        x1_my = x0q_ref[my] + bsel(ga_, my) * attn_my
        x1ag_ref[my] = x1_my.astype(BF16)
        ag1 = exchange(
            1,
            lambda off, dev: x1ag_ref.at[my],
            lambda off, dev: x1ag_ref.at[my],
        )

        loads[7].wait()
        loads[8].wait()
        wff1_b = wff1_ref[...].astype(BF16)
        wff2_b = wff2_ref[...].astype(BF16)

        def ffn_block(x1_blk, qidx):
            xm2 = ln_mod(x1_blk, bsel(sm_, qidx), bsel(shm_, qidx))
            hb = jnp.dot(xm2.astype(BF16), wff1_b, preferred_element_type=F32)
            hb = hb / (1.0 + jnp.exp(-hb))
            return jnp.dot(hb.astype(BF16), wff2_b, preferred_element_type=F32)

        ffn_my = ffn_block(x1_my, my)

        rs2 = []
        for off in (1, 2, 3):
            ag1[off - 1].wait_recv()
            qidx = lax.rem(my - off + N_DEV, N_DEV)
            fblk = ffn_block(x1ag_ref[qidx].astype(F32), qidx)
            p2_ref[qidx] = fblk.astype(BF16)
            off_send = N_DEV - off
            r = mk_rdma(
                2, off_send, p2_ref.at[qidx], rs2_ref.at[off_send - 1], qidx
            )
            r.start()
            rs2.append(r)

        for r in rs2:
            r.wait_recv()
        total2 = ffn_my
        for i in range(N_DEV - 1):
            total2 = total2 + rs2_ref[i].astype(F32)
        out_my = x1_my + bsel(gm_, my) * total2
        red2_ref[my] = out_my.astype(BF16)
        ag2 = exchange(
            3,
            lambda off, dev: red2_ref.at[my],
            lambda off, dev: red2_ref.at[my],
        )
        for r in ag2:
            r.wait_recv()

        for b in range(B):
            out_ref[b, :, :] = jnp.concatenate(
                [red2_ref[2 * b].astype(F32), red2_ref[2 * b + 1].astype(F32)],
                axis=0,
            )

        for r in ag1:
            r.wait_send()
        for r in rs2:
            r.wait_send()
        for r in ag2:
            r.wait_send()

    return pl.pallas_call(
        body,
        out_shape=jax.ShapeDtypeStruct((B, SQ, D), jnp.float32),
        in_specs=[pl.BlockSpec(memory_space=pltpu.MemorySpace.HBM)] * 9,
        out_specs=pl.BlockSpec(memory_space=pltpu.VMEM),
        scratch_shapes=[
            pltpu.VMEM((B, SQ, D), F32),
            pltpu.VMEM((D, 256), F32),
            pltpu.VMEM((D, 256), F32),
            pltpu.VMEM((D, 256), F32),
            pltpu.VMEM((256, D), F32),
            pltpu.VMEM((B, 128), F32),
            pltpu.VMEM((128, 6 * D), F32),
            pltpu.VMEM((D, D), F32),
            pltpu.VMEM((D, D), F32),
            pltpu.VMEM((N_DEV, Q4, D), F32),
            pltpu.VMEM((N_DEV, Q4, D), F32),
            pltpu.VMEM((N_DEV, Q4, D), BF16),
            pltpu.VMEM((3, Q4, D), BF16),
            pltpu.VMEM((N_DEV, Q4, D), BF16),
            pltpu.VMEM((N_DEV, Q4, D), BF16),
            pltpu.VMEM((3, Q4, D), BF16),
            pltpu.VMEM((N_DEV, Q4, D), BF16),
            pltpu.SemaphoreType.DMA((9,)),
            pltpu.SemaphoreType.DMA((12,)),
            pltpu.SemaphoreType.DMA((12,)),
        ],
        compiler_params=pltpu.CompilerParams(collective_id=0),
    )(x, Wq, Wk, Wv, Wo, t_emb, W_mod, W_ff1, W_ff2)


# device time: 28322 ns/iter; 1.3521x vs baseline; 1.3515x over previous
import jax
import jax.numpy as jnp
from jax import lax
from jax.experimental import pallas as pl
from jax.experimental.pallas import tpu as pltpu

N_DEV = 4
B, SQ, D = 2, 256, 512
H_LOCAL, DH = 4, 64
ROWS = B * SQ
Q4 = ROWS // N_DEV
EPS = 1e-5
F32 = jnp.float32
BF16 = jnp.bfloat16


def kernel(x, Wq, Wk, Wv, Wo, t_emb, W_mod, W_ff1, W_ff2):
    def body(
        x_hbm, wq_hbm, wk_hbm, wv_hbm, wo_hbm, temb_hbm, wmod_hbm,
        wff1_hbm, wff2_hbm, out_ref,
        xv_ref, wq_ref, wk_ref, wv_ref, wo_ref, temb_ref, wmod_ref,
        wff1_ref, wff2_ref,
        x0q_ref,
        pf_ref,
        p1_ref,
        rs1_ref,
        x1ag_ref,
        p2_ref,
        rs2_ref,
        red2_ref,
        load_sems,
        send_sems, recv_sems,
    ):
        my = lax.axis_index("i")

        hbm = [x_hbm, temb_hbm, wmod_hbm, wq_hbm, wk_hbm, wv_hbm,
               wo_hbm, wff1_hbm, wff2_hbm]
        vmem = [xv_ref, temb_ref, wmod_ref, wq_ref, wk_ref, wv_ref,
                wo_ref, wff1_ref, wff2_ref]
        loads = []
        for i, (h, v) in enumerate(zip(hbm, vmem)):
            c = pltpu.make_async_copy(h, v, load_sems.at[i])
            c.start()
            loads.append(c)

        barrier_sem = pltpu.get_barrier_semaphore()
        for off in (1, 2, 3):
            pl.semaphore_signal(
                barrier_sem, inc=1,
                device_id=(lax.rem(my + off, N_DEV),),
                device_id_type=pl.DeviceIdType.MESH,
            )
        pl.semaphore_wait(barrier_sem, N_DEV - 1)

        def mk_rdma(phase, off, src, dst, dev):
            return pltpu.make_async_remote_copy(
                src_ref=src,
                dst_ref=dst,
                send_sem=send_sems.at[phase * 3 + off - 1],
                recv_sem=recv_sems.at[phase * 3 + off - 1],
                device_id=(dev,),
                device_id_type=pl.DeviceIdType.MESH,
            )

        def exchange(phase, make_src, make_dst):
            rdmas = []
            for off in (1, 2, 3):
                dev = lax.rem(my + off, N_DEV)
                rdma = mk_rdma(phase, off, make_src(off, dev),
                               make_dst(off, dev), dev)
                rdma.start()
                rdmas.append(rdma)
            return rdmas

        def store_quarters(ref, val):
            for q in range(N_DEV):
                ref[q] = val[q * Q4 : (q + 1) * Q4, :].astype(ref.dtype)

        def bsel(pair, qidx):
            return jnp.where(qidx >= 2, pair[1], pair[0])

        loads[1].wait()
        loads[2].wait()
        mod = []
        for b in range(B):
            mb = jnp.dot(
                temb_ref[b : b + 1, :], wmod_ref[...],
                preferred_element_type=F32,
            )
            mod.append([mb[:, i * D : (i + 1) * D] for i in range(6)])
        sa_, sha_, ga_, sm_, shm_, gm_ = (
            [mod[0][i], mod[1][i]] for i in range(6)
        )

        loads[0].wait()
        for q in range(N_DEV):
            x0q_ref[q] = xv_ref[q // 2][(q % 2) * Q4 : (q % 2 + 1) * Q4, :]

        def ln_mod(h, scale, shift):
            m = jnp.mean(h, axis=-1, keepdims=True)
            v = jnp.mean((h - m) * (h - m), axis=-1, keepdims=True)
            return ((h - m) * lax.rsqrt(v + EPS)) * (1.0 + scale) + shift

        loads[3].wait()
        loads[4].wait()
        loads[5].wait()
        wq_b = wq_ref[...].astype(BF16)
        wk_b = wk_ref[...].astype(BF16)
        wv_b = wv_ref[...].astype(BF16)
        attn_parts = []
        for b in range(B):
            xm = ln_mod(xv_ref[b], sa_[b], sha_[b]).astype(BF16)
            q = jnp.dot(xm, wq_b, preferred_element_type=F32).astype(BF16)
            k = jnp.dot(xm, wk_b, preferred_element_type=F32).astype(BF16)
            v = jnp.dot(xm, wv_b, preferred_element_type=F32).astype(BF16)
            outs = []
            for h in range(H_LOCAL):
                sl = slice(h * DH, (h + 1) * DH)
                s = lax.dot_general(
                    q[:, sl], k[:, sl], (((1,), (1,)), ((), ())),
                    preferred_element_type=F32,
                ) * 0.125
                mx = jnp.max(s, axis=-1, keepdims=True)
                p = jnp.exp(s - mx)
                l = jnp.sum(p, axis=-1, keepdims=True)
                outs.append(
                    jnp.dot(
                        p.astype(BF16), v[:, sl], preferred_element_type=F32
                    ) / l
                )
            o = jnp.concatenate(outs, axis=1).astype(BF16)
            if b == 0:
                loads[6].wait()
                wo_b = wo_ref[...].astype(BF16)
            attn_parts.append(jnp.dot(o, wo_b, preferred_element_type=F32))
        attn_partial = jnp.concatenate(attn_parts, axis=0)
        store_quarters(pf_ref, attn_partial)
        store_quarters(p1_ref, attn_partial)

        rs1 = exchange(
            0,
            lambda off, dev: p1_ref.at[dev],
            lambda off, dev: rs1_ref.at[off - 1],
        )
        for r in rs1:
            r.wait()
        attn_my = pf_ref[my]
        for i in range(N_DEV - 1):
            attn_my = attn_my + rs1_ref[i].astype(F32)

        x1_my = x0q_ref[my] + bsel(ga_, my) * attn_my
        x1ag_ref[my] = x1_my.astype(BF16)
        ag1 = exchange(
            1,
            lambda off, dev: x1ag_ref.at[my],
            lambda off, dev: x1ag_ref.at[my],
        )

        loads[7].wait()
        loads[8].wait()
        wff1_b = wff1_ref[...].astype(BF16)
        wff2_b = wff2_ref[...].astype(BF16)

        def ffn_block(x1_blk, qidx):
            xm2 = ln_mod(x1_blk, bsel(sm_, qidx), bsel(shm_, qidx))
            hb = jnp.dot(xm2.astype(BF16), wff1_b, preferred_element_type=F32)
            hb = hb / (1.0 + jnp.exp(-hb))
            return jnp.dot(hb.astype(BF16), wff2_b, preferred_element_type=F32)

        ffn_my = ffn_block(x1_my, my)

        rs2 = []
        for off in (1, 2, 3):
            ag1[off - 1].wait_recv()
            qidx = lax.rem(my - off + N_DEV, N_DEV)
            fblk = ffn_block(x1ag_ref[qidx].astype(F32), qidx)
            p2_ref[qidx] = fblk.astype(BF16)
            off_send = N_DEV - off
            r = mk_rdma(
                2, off_send, p2_ref.at[qidx], rs2_ref.at[off_send - 1], qidx
            )
            r.start()
            rs2.append(r)

        for r in rs2:
            r.wait_recv()
        total2 = ffn_my
        for i in range(N_DEV - 1):
            total2 = total2 + rs2_ref[i].astype(F32)
        out_my = x1_my + bsel(gm_, my) * total2
        red2_ref[my] = out_my.astype(BF16)
        ag2 = exchange(
            3,
            lambda off, dev: red2_ref.at[my],
            lambda off, dev: red2_ref.at[my],
        )
        for r in ag2:
            r.wait_recv()

        for b in range(B):
            out_ref[b, :, :] = jnp.concatenate(
                [red2_ref[2 * b].astype(F32), red2_ref[2 * b + 1].astype(F32)],
                axis=0,
            )

        for r in ag1:
            r.wait_send()
        for r in rs2:
            r.wait_send()
        for r in ag2:
            r.wait_send()

    return pl.pallas_call(
        body,
        out_shape=jax.ShapeDtypeStruct((B, SQ, D), jnp.float32),
        in_specs=[pl.BlockSpec(memory_space=pltpu.MemorySpace.HBM)] * 9,
        out_specs=pl.BlockSpec(memory_space=pltpu.VMEM),
        scratch_shapes=[
            pltpu.VMEM((B, SQ, D), F32),
            pltpu.VMEM((D, 256), F32),
            pltpu.VMEM((D, 256), F32),
            pltpu.VMEM((D, 256), F32),
            pltpu.VMEM((256, D), F32),
            pltpu.VMEM((B, 128), F32),
            pltpu.VMEM((128, 6 * D), F32),
            pltpu.VMEM((D, D), F32),
            pltpu.VMEM((D, D), F32),
            pltpu.VMEM((N_DEV, Q4, D), F32),
            pltpu.VMEM((N_DEV, Q4, D), F32),
            pltpu.VMEM((N_DEV, Q4, D), BF16),
            pltpu.VMEM((3, Q4, D), BF16),
            pltpu.VMEM((N_DEV, Q4, D), BF16),
            pltpu.VMEM((N_DEV, Q4, D), BF16),
            pltpu.VMEM((3, Q4, D), BF16),
            pltpu.VMEM((N_DEV, Q4, D), BF16),
            pltpu.SemaphoreType.DMA((9,)),
            pltpu.SemaphoreType.DMA((12,)),
            pltpu.SemaphoreType.DMA((12,)),
        ],
        compiler_params=pltpu.CompilerParams(collective_id=0),
    )(*(
        pltpu.with_memory_space_constraint(a, pltpu.MemorySpace.HBM)
        for a in (x, Wq, Wk, Wv, Wo, t_emb, W_mod, W_ff1, W_ff2)
    ))


# device time: 27822 ns/iter; 1.3764x vs baseline; 1.0180x over previous
import jax
import jax.numpy as jnp
from jax import lax
from jax.experimental import pallas as pl
from jax.experimental.pallas import tpu as pltpu

N_DEV = 4
B, SQ, D = 2, 256, 512
H_LOCAL, DH = 4, 64
ROWS = B * SQ
Q4 = ROWS // N_DEV
EPS = 1e-5
F32 = jnp.float32
BF16 = jnp.bfloat16


def kernel(x, Wq, Wk, Wv, Wo, t_emb, W_mod, W_ff1, W_ff2):
    def body(
        x_hbm, wq_hbm, wk_hbm, wv_hbm, wo_hbm, temb_hbm, wmod_hbm,
        wff1_hbm, wff2_hbm, out_ref,
        xv_ref, wq_ref, wk_ref, wv_ref, wo_ref, temb_ref, wmod_ref,
        wff1_ref, wff2_ref,
        x0q_ref,
        pf_ref,
        p1_ref,
        rs1_ref,
        x1ag_ref,
        p2_ref,
        rs2_ref,
        red2_ref,
        load_sems,
        send_sems, recv_sems,
    ):
        my = lax.axis_index("i")

        hbm = [x_hbm, temb_hbm, wmod_hbm, wq_hbm, wk_hbm, wv_hbm,
               wo_hbm, wff1_hbm, wff2_hbm]
        vmem = [xv_ref, temb_ref, wmod_ref, wq_ref, wk_ref, wv_ref,
                wo_ref, wff1_ref, wff2_ref]
        loads = []
        for i, (h, v) in enumerate(zip(hbm, vmem)):
            c = pltpu.make_async_copy(h, v, load_sems.at[i])
            c.start()
            loads.append(c)

        barrier_sem = pltpu.get_barrier_semaphore()
        for off in (1, 2, 3):
            pl.semaphore_signal(
                barrier_sem, inc=1,
                device_id=(lax.rem(my + off, N_DEV),),
                device_id_type=pl.DeviceIdType.MESH,
            )
        pl.semaphore_wait(barrier_sem, N_DEV - 1)

        def mk_rdma(phase, off, src, dst, dev, sem_idx=None):
            idx = phase * 3 + off - 1 if sem_idx is None else sem_idx
            return pltpu.make_async_remote_copy(
                src_ref=src,
                dst_ref=dst,
                send_sem=send_sems.at[idx],
                recv_sem=recv_sems.at[idx],
                device_id=(dev,),
                device_id_type=pl.DeviceIdType.MESH,
            )

        def exchange(phase, make_src, make_dst):
            rdmas = []
            for off in (1, 2, 3):
                dev = lax.rem(my + off, N_DEV)
                rdma = mk_rdma(phase, off, make_src(off, dev),
                               make_dst(off, dev), dev)
                rdma.start()
                rdmas.append(rdma)
            return rdmas

        def bsel(pair, qidx):
            return jnp.where(qidx >= 2, pair[1], pair[0])

        loads[0].wait()
        for q in range(N_DEV):
            x0q_ref[q] = xv_ref[q // 2][(q % 2) * Q4 : (q % 2 + 1) * Q4, :]
        ln1_stats = []
        for b in range(B):
            h = xv_ref[b]
            m = jnp.mean(h, axis=-1, keepdims=True)
            v = jnp.mean((h - m) * (h - m), axis=-1, keepdims=True)
            ln1_stats.append((h - m) * lax.rsqrt(v + EPS))

        loads[1].wait()
        loads[2].wait()
        mod = []
        for b in range(B):
            mb = jnp.dot(
                temb_ref[b : b + 1, :], wmod_ref[...],
                preferred_element_type=F32,
            )
            mod.append([mb[:, i * D : (i + 1) * D] for i in range(6)])
        sa_, sha_, ga_, sm_, shm_, gm_ = (
            [mod[0][i], mod[1][i]] for i in range(6)
        )

        def ln_mod(h, scale, shift):
            m = jnp.mean(h, axis=-1, keepdims=True)
            v = jnp.mean((h - m) * (h - m), axis=-1, keepdims=True)
            return ((h - m) * lax.rsqrt(v + EPS)) * (1.0 + scale) + shift

        loads[3].wait()
        loads[4].wait()
        loads[5].wait()
        wq_b = wq_ref[...].astype(BF16)
        wk_b = wk_ref[...].astype(BF16)
        wv_b = wv_ref[...].astype(BF16)
        rs1_sends = []
        for b in range(B):
            xm = (ln1_stats[b] * (1.0 + sa_[b]) + sha_[b]).astype(BF16)
            q = jnp.dot(xm, wq_b, preferred_element_type=F32).astype(BF16)
            k = jnp.dot(xm, wk_b, preferred_element_type=F32).astype(BF16)
            v = jnp.dot(xm, wv_b, preferred_element_type=F32).astype(BF16)
            outs = []
            for h in range(H_LOCAL):
                sl = slice(h * DH, (h + 1) * DH)
                s = lax.dot_general(
                    q[:, sl], k[:, sl], (((1,), (1,)), ((), ())),
                    preferred_element_type=F32,
                ) * 0.125
                mx = jnp.max(s, axis=-1, keepdims=True)
                p = jnp.exp(s - mx)
                l = jnp.sum(p, axis=-1, keepdims=True)
                outs.append(
                    jnp.dot(
                        p.astype(BF16), v[:, sl], preferred_element_type=F32
                    ) / l
                )
            o = jnp.concatenate(outs, axis=1).astype(BF16)
            if b == 0:
                loads[6].wait()
                wo_b = wo_ref[...].astype(BF16)
            part = jnp.dot(o, wo_b, preferred_element_type=F32)
            for qq in (2 * b, 2 * b + 1):
                blk = part[(qq - 2 * b) * Q4 : (qq - 2 * b + 1) * Q4, :]
                pf_ref[qq] = blk
                p1_ref[qq] = blk.astype(BF16)
                idx = jnp.maximum(lax.rem(qq - my + N_DEV, N_DEV) - 1, 0)
                r = mk_rdma(
                    0, 0, p1_ref.at[qq], rs1_ref.at[idx], qq, sem_idx=idx
                )
                @pl.when(my != qq)
                def _():
                    r.start()
                rs1_sends.append((r, qq))

        rs1_waits = [
            mk_rdma(0, off, p1_ref.at[0], rs1_ref.at[off - 1],
                    lax.rem(my + off, N_DEV))
            for off in (1, 2, 3)
        ]
        for r in rs1_waits:
            r.wait_recv()
        attn_my = pf_ref[my]
        for i in range(N_DEV - 1):
            attn_my = attn_my + rs1_ref[i].astype(F32)

        x1_my = x0q_ref[my] + bsel(ga_, my) * attn_my
        x1ag_ref[my] = x1_my.astype(BF16)
        ag1 = exchange(
            1,
            lambda off, dev: x1ag_ref.at[my],
            lambda off, dev: x1ag_ref.at[my],
        )

        loads[7].wait()
        loads[8].wait()
        wff1_b = wff1_ref[...].astype(BF16)
        wff2_b = wff2_ref[...].astype(BF16)

        def ffn_block(x1_blk, qidx):
            xm2 = ln_mod(x1_blk, bsel(sm_, qidx), bsel(shm_, qidx))
            hb = jnp.dot(xm2.astype(BF16), wff1_b, preferred_element_type=F32)
            hb = hb / (1.0 + jnp.exp(-hb))
            return jnp.dot(hb.astype(BF16), wff2_b, preferred_element_type=F32)

        ffn_my = ffn_block(x1_my, my)

        rs2 = []
        for off in (1, 2, 3):
            ag1[off - 1].wait_recv()
            qidx = lax.rem(my - off + N_DEV, N_DEV)
            fblk = ffn_block(x1ag_ref[qidx].astype(F32), qidx)
            p2_ref[qidx] = fblk.astype(BF16)
            off_send = N_DEV - off
            r = mk_rdma(
                2, off_send, p2_ref.at[qidx], rs2_ref.at[off_send - 1], qidx
            )
            r.start()
            rs2.append(r)

        for r in rs2:
            r.wait_recv()
        total2 = ffn_my
        for i in range(N_DEV - 1):
            total2 = total2 + rs2_ref[i].astype(F32)
        out_my = x1_my + bsel(gm_, my) * total2
        red2_ref[my] = out_my.astype(BF16)
        ag2 = exchange(
            3,
            lambda off, dev: red2_ref.at[my],
            lambda off, dev: red2_ref.at[my],
        )
        out_ref[my // 2, pl.ds(lax.rem(my, 2) * Q4, Q4), :] = out_my
        for off in (1, 2, 3):
            ag2[off - 1].wait_recv()
            qidx = lax.rem(my - off + N_DEV, N_DEV)
            out_ref[qidx // 2, pl.ds(lax.rem(qidx, 2) * Q4, Q4), :] = (
                red2_ref[qidx].astype(F32)
            )

        for r, qq in rs1_sends:
            @pl.when(my != qq)
            def _():
                r.wait_send()
        for r in ag1:
            r.wait_send()
        for r in rs2:
            r.wait_send()
        for r in ag2:
            r.wait_send()

    return pl.pallas_call(
        body,
        out_shape=jax.ShapeDtypeStruct((B, SQ, D), jnp.float32),
        in_specs=[pl.BlockSpec(memory_space=pltpu.MemorySpace.HBM)] * 9,
        out_specs=pl.BlockSpec(memory_space=pltpu.MemorySpace.VMEM),
        scratch_shapes=[
            pltpu.VMEM((B, SQ, D), F32),
            pltpu.VMEM((D, 256), F32),
            pltpu.VMEM((D, 256), F32),
            pltpu.VMEM((D, 256), F32),
            pltpu.VMEM((256, D), F32),
            pltpu.VMEM((B, 128), F32),
            pltpu.VMEM((128, 6 * D), F32),
            pltpu.VMEM((D, D), F32),
            pltpu.VMEM((D, D), F32),
            pltpu.VMEM((N_DEV, Q4, D), F32),
            pltpu.VMEM((N_DEV, Q4, D), F32),
            pltpu.VMEM((N_DEV, Q4, D), BF16),
            pltpu.VMEM((3, Q4, D), BF16),
            pltpu.VMEM((N_DEV, Q4, D), BF16),
            pltpu.VMEM((N_DEV, Q4, D), BF16),
            pltpu.VMEM((3, Q4, D), BF16),
            pltpu.VMEM((N_DEV, Q4, D), BF16),
            pltpu.SemaphoreType.DMA((9,)),
            pltpu.SemaphoreType.DMA((12,)),
            pltpu.SemaphoreType.DMA((12,)),
        ],
        compiler_params=pltpu.CompilerParams(collective_id=0),
    )(*(
        pltpu.with_memory_space_constraint(a, pltpu.MemorySpace.HBM)
        for a in (x, Wq, Wk, Wv, Wo, t_emb, W_mod, W_ff1, W_ff2)
    ))


# device time: 27500 ns/iter; 1.3925x vs baseline; 1.0117x over previous
import jax
import jax.numpy as jnp
from jax import lax
from jax.experimental import pallas as pl
from jax.experimental.pallas import tpu as pltpu

N_DEV = 4
B, SQ, D = 2, 256, 512
H_LOCAL, DH = 4, 64
ROWS = B * SQ
Q4 = ROWS // N_DEV
EPS = 1e-5
F32 = jnp.float32
BF16 = jnp.bfloat16


def kernel(x, Wq, Wk, Wv, Wo, t_emb, W_mod, W_ff1, W_ff2):
    def body(
        x_hbm, wq_hbm, wk_hbm, wv_hbm, wo_hbm, temb_hbm, wmod_hbm,
        wff1_hbm, wff2_hbm, out_ref,
        xv_ref, wq_ref, wk_ref, wv_ref, wo_ref, temb_ref, wmod_ref,
        wff1_ref, wff2_ref,
        x0q_ref,
        pf_ref,
        p1_ref,
        rs1_ref,
        x1ag_ref,
        p2_ref,
        rs2_ref,
        red2_ref,
        outv_ref,
        load_sems,
        out_sems,
        send_sems, recv_sems,
    ):
        my = lax.axis_index("i")

        hbm = [x_hbm, temb_hbm, wmod_hbm, wq_hbm, wk_hbm, wv_hbm,
               wo_hbm, wff1_hbm, wff2_hbm]
        vmem = [xv_ref, temb_ref, wmod_ref, wq_ref, wk_ref, wv_ref,
                wo_ref, wff1_ref, wff2_ref]
        loads = []
        for i, (h, v) in enumerate(zip(hbm, vmem)):
            c = pltpu.make_async_copy(h, v, load_sems.at[i])
            c.start()
            loads.append(c)

        barrier_sem = pltpu.get_barrier_semaphore()
        for off in (1, 2, 3):
            pl.semaphore_signal(
                barrier_sem, inc=1,
                device_id=(lax.rem(my + off, N_DEV),),
                device_id_type=pl.DeviceIdType.MESH,
            )
        pl.semaphore_wait(barrier_sem, N_DEV - 1)

        def mk_rdma(phase, off, src, dst, dev, sem_idx=None):
            idx = phase * 3 + off - 1 if sem_idx is None else sem_idx
            return pltpu.make_async_remote_copy(
                src_ref=src,
                dst_ref=dst,
                send_sem=send_sems.at[idx],
                recv_sem=recv_sems.at[idx],
                device_id=(dev,),
                device_id_type=pl.DeviceIdType.MESH,
            )

        def exchange(phase, make_src, make_dst):
            rdmas = []
            for off in (1, 2, 3):
                dev = lax.rem(my + off, N_DEV)
                rdma = mk_rdma(phase, off, make_src(off, dev),
                               make_dst(off, dev), dev)
                rdma.start()
                rdmas.append(rdma)
            return rdmas

        def bsel(pair, qidx):
            return jnp.where(qidx >= 2, pair[1], pair[0])

        loads[0].wait()
        for q in range(N_DEV):
            x0q_ref[q] = xv_ref[q // 2][(q % 2) * Q4 : (q % 2 + 1) * Q4, :]
        ln1_stats = []
        for b in range(B):
            h = xv_ref[b]
            m = jnp.mean(h, axis=-1, keepdims=True)
            v = jnp.mean((h - m) * (h - m), axis=-1, keepdims=True)
            ln1_stats.append((h - m) * lax.rsqrt(v + EPS))

        loads[1].wait()
        loads[2].wait()
        mod = []
        for b in range(B):
            mb = jnp.dot(
                temb_ref[b : b + 1, :], wmod_ref[...],
                preferred_element_type=F32,
            )
            mod.append([mb[:, i * D : (i + 1) * D] for i in range(6)])
        sa_, sha_, ga_, sm_, shm_, gm_ = (
            [mod[0][i], mod[1][i]] for i in range(6)
        )

        def ln_mod(h, scale, shift):
            m = jnp.mean(h, axis=-1, keepdims=True)
            v = jnp.mean((h - m) * (h - m), axis=-1, keepdims=True)
            return ((h - m) * lax.rsqrt(v + EPS)) * (1.0 + scale) + shift

        loads[3].wait()
        loads[4].wait()
        loads[5].wait()
        wq_b = wq_ref[...].astype(BF16)
        wk_b = wk_ref[...].astype(BF16)
        wv_b = wv_ref[...].astype(BF16)
        rs1_sends = []
        for b in range(B):
            xm = (ln1_stats[b] * (1.0 + sa_[b]) + sha_[b]).astype(BF16)
            q = jnp.dot(xm, wq_b, preferred_element_type=F32).astype(BF16)
            k = jnp.dot(xm, wk_b, preferred_element_type=F32).astype(BF16)
            v = jnp.dot(xm, wv_b, preferred_element_type=F32).astype(BF16)
            outs = []
            for h in range(H_LOCAL):
                sl = slice(h * DH, (h + 1) * DH)
                s = lax.dot_general(
                    q[:, sl], k[:, sl], (((1,), (1,)), ((), ())),
                    preferred_element_type=F32,
                ) * 0.125
                mx = jnp.max(s, axis=-1, keepdims=True)
                p = jnp.exp(s - mx)
                l = jnp.sum(p, axis=-1, keepdims=True)
                outs.append(
                    jnp.dot(
                        p.astype(BF16), v[:, sl], preferred_element_type=F32
                    ) / l
                )
            o = jnp.concatenate(outs, axis=1).astype(BF16)
            if b == 0:
                loads[6].wait()
                wo_b = wo_ref[...].astype(BF16)
            part = jnp.dot(o, wo_b, preferred_element_type=F32)
            for qq in (2 * b, 2 * b + 1):
                blk = part[(qq - 2 * b) * Q4 : (qq - 2 * b + 1) * Q4, :]
                pf_ref[qq] = blk
                p1_ref[qq] = blk.astype(BF16)
                idx = jnp.maximum(lax.rem(qq - my + N_DEV, N_DEV) - 1, 0)
                r = mk_rdma(
                    0, 0, p1_ref.at[qq], rs1_ref.at[idx], qq, sem_idx=idx
                )
                @pl.when(my != qq)
                def _():
                    r.start()
                rs1_sends.append((r, qq))

        rs1_waits = [
            mk_rdma(0, off, p1_ref.at[0], rs1_ref.at[off - 1],
                    lax.rem(my + off, N_DEV))
            for off in (1, 2, 3)
        ]
        for r in rs1_waits:
            r.wait_recv()
        attn_my = pf_ref[my]
        for i in range(N_DEV - 1):
            attn_my = attn_my + rs1_ref[i].astype(F32)

        x1_my = x0q_ref[my] + bsel(ga_, my) * attn_my
        x1ag_ref[my] = x1_my.astype(BF16)
        ag1 = exchange(
            1,
            lambda off, dev: x1ag_ref.at[my],
            lambda off, dev: x1ag_ref.at[my],
        )

        loads[7].wait()
        loads[8].wait()
        wff1_b = wff1_ref[...].astype(BF16)
        wff2_b = wff2_ref[...].astype(BF16)

        def ffn_block(x1_blk, qidx):
            xm2 = ln_mod(x1_blk, bsel(sm_, qidx), bsel(shm_, qidx))
            hb = jnp.dot(xm2.astype(BF16), wff1_b, preferred_element_type=F32)
            hb = hb / (1.0 + jnp.exp(-hb))
            return jnp.dot(hb.astype(BF16), wff2_b, preferred_element_type=F32)

        ffn_my = ffn_block(x1_my, my)

        HD = D // 2
        halves = (slice(0, HD), slice(HD, D))
        rs2 = []
        for off in (1, 2, 3):
            ag1[off - 1].wait_recv()
            qidx = lax.rem(my - off + N_DEV, N_DEV)
            fblk = ffn_block(x1ag_ref[qidx].astype(F32), qidx)
            p2_ref[qidx] = fblk.astype(BF16)
            off_send = N_DEV - off
            pair = []
            for hf in (0, 1):
                r = mk_rdma(
                    0, 0,
                    p2_ref.at[qidx, :, pl.ds(hf * HD, HD)],
                    rs2_ref.at[off_send - 1, :, pl.ds(hf * HD, HD)],
                    qidx, sem_idx=12 + (off_send - 1) * 2 + hf,
                )
                r.start()
                pair.append(r)
            rs2.append(pair)

        ag2 = []
        out_half = []
        for hf in (0, 1):
            for r in rs2:
                r[hf].wait_recv()
            total2 = ffn_my[:, halves[hf]]
            for i in range(N_DEV - 1):
                total2 = total2 + rs2_ref[i, :, halves[hf]].astype(F32)
            oh = (
                x1_my[:, halves[hf]]
                + bsel(gm_, my)[:, halves[hf]] * total2
            )
            out_half.append(oh)
            red2_ref[my, :, halves[hf]] = oh.astype(BF16)
            for off in (1, 2, 3):
                dev = lax.rem(my + off, N_DEV)
                r = mk_rdma(
                    0, 0,
                    red2_ref.at[my, :, pl.ds(hf * HD, HD)],
                    red2_ref.at[my, :, pl.ds(hf * HD, HD)],
                    dev, sem_idx=18 + (off - 1) * 2 + hf,
                )
                r.start()
                ag2.append(r)
        outv_ref[my] = jnp.concatenate(out_half, axis=1)
        out_dmas = []
        c = pltpu.make_async_copy(
            outv_ref.at[my],
            out_ref.at[my // 2, pl.ds(lax.rem(my, 2) * Q4, Q4), :],
            out_sems.at[0],
        )
        c.start()
        out_dmas.append(c)
        for off in (1, 2, 3):
            for hf in (0, 1):
                ag2[hf * 3 + (off - 1)].wait_recv()
            qidx = lax.rem(my - off + N_DEV, N_DEV)
            outv_ref[qidx] = red2_ref[qidx].astype(F32)
            c = pltpu.make_async_copy(
                outv_ref.at[qidx],
                out_ref.at[qidx // 2, pl.ds(lax.rem(qidx, 2) * Q4, Q4), :],
                out_sems.at[off],
            )
            c.start()
            out_dmas.append(c)
        for c in out_dmas:
            c.wait()

        for r, qq in rs1_sends:
            @pl.when(my != qq)
            def _():
                r.wait_send()
        for r in ag1:
            r.wait_send()
        for pair in rs2:
            for r in pair:
                r.wait_send()
        for r in ag2:
            r.wait_send()

    return pl.pallas_call(
        body,
        out_shape=jax.ShapeDtypeStruct((B, SQ, D), jnp.float32),
        in_specs=[pl.BlockSpec(memory_space=pltpu.MemorySpace.HBM)] * 9,
        out_specs=pl.BlockSpec(memory_space=pltpu.MemorySpace.HBM),
        scratch_shapes=[
            pltpu.VMEM((B, SQ, D), F32),
            pltpu.VMEM((D, 256), F32),
            pltpu.VMEM((D, 256), F32),
            pltpu.VMEM((D, 256), F32),
            pltpu.VMEM((256, D), F32),
            pltpu.VMEM((B, 128), F32),
            pltpu.VMEM((128, 6 * D), F32),
            pltpu.VMEM((D, D), F32),
            pltpu.VMEM((D, D), F32),
            pltpu.VMEM((N_DEV, Q4, D), F32),
            pltpu.VMEM((N_DEV, Q4, D), F32),
            pltpu.VMEM((N_DEV, Q4, D), BF16),
            pltpu.VMEM((3, Q4, D), BF16),
            pltpu.VMEM((N_DEV, Q4, D), BF16),
            pltpu.VMEM((N_DEV, Q4, D), BF16),
            pltpu.VMEM((3, Q4, D), BF16),
            pltpu.VMEM((N_DEV, Q4, D), BF16),
            pltpu.VMEM((N_DEV, Q4, D), F32),
            pltpu.SemaphoreType.DMA((9,)),
            pltpu.SemaphoreType.DMA((4,)),
            pltpu.SemaphoreType.DMA((24,)),
            pltpu.SemaphoreType.DMA((24,)),
        ],
        compiler_params=pltpu.CompilerParams(collective_id=0),
    )(*(
        pltpu.with_memory_space_constraint(a, pltpu.MemorySpace.HBM)
        for a in (x, Wq, Wk, Wv, Wo, t_emb, W_mod, W_ff1, W_ff2)
    ))
